# Initial kernel scaffold; baseline (speedup 1.0000x reference)
import functools

import jax
import jax.numpy as jnp
from jax import lax
from jax.experimental import pallas as pl
from jax.experimental.pallas import tpu as pltpu
from jax.experimental.pallas import tpu_sc as plsc

DIM = 32
NC = 2
NS = 16
NW = NC * NS
SUB = 128
CHUNK = 1024


def _emb_lookup(table, idx2d, *, B, per_w, n_chunks):
    mesh = plsc.VectorSubcoreMesh(core_axis_name="c", subcore_axis_name="s")

    @functools.partial(
        pl.kernel,
        mesh=mesh,
        out_type=jax.ShapeDtypeStruct((B, DIM), jnp.float32),
        scratch_types=[
            pltpu.VMEM((CHUNK // SUB, SUB), jnp.int32),
            pltpu.VMEM((CHUNK, DIM), jnp.float32),
            pltpu.SemaphoreType.DMA,
        ],
    )
    def _k(table_hbm, idx_hbm, out_hbm, idx_v, rows_v, sem):
        wid = lax.axis_index("s") * NC + lax.axis_index("c")
        idx_row_base = wid * (per_w // SUB)
        out_base = wid * per_w

        def body(g, carry):
            pltpu.sync_copy(
                idx_hbm.at[pl.ds(idx_row_base + g * (CHUNK // SUB), CHUNK // SUB)],
                idx_v,
            )
            copies = [
                pltpu.async_copy(
                    table_hbm.at[idx_v.at[j]],
                    rows_v.at[pl.ds(j * SUB, SUB)],
                    sem,
                )
                for j in range(CHUNK // SUB)
            ]
            for c in copies:
                c.wait()
            pltpu.sync_copy(rows_v, out_hbm.at[pl.ds(out_base + g * CHUNK, CHUNK)])
            return carry

        lax.fori_loop(0, n_chunks, body, 0)

    return _k(table, idx2d)


def kernel(inputs, embeddings):
    shape = inputs.shape
    flat = jnp.reshape(inputs, (-1,)).astype(jnp.int32)
    B = flat.shape[0]
    per_w = B // NW
    n_chunks = per_w // CHUNK
    idx2d = jnp.reshape(flat, (B // SUB, SUB))
    out = _emb_lookup(embeddings, idx2d, B=B, per_w=per_w, n_chunks=n_chunks)
    return jnp.reshape(out, shape + (DIM,))


# speedup vs baseline: 4.8064x; 4.8064x over previous
import functools

import jax
import jax.numpy as jnp
from jax import lax
from jax.experimental import pallas as pl
from jax.experimental.pallas import tpu as pltpu
from jax.experimental.pallas import tpu_sc as plsc

DIM = 32
NC = 2
NS = 16
NW = NC * NS
SUB = 128
CHUNK = 1024


def _emb_lookup(table, idx2d, *, B, per_w, n_chunks):
    mesh = plsc.VectorSubcoreMesh(core_axis_name="c", subcore_axis_name="s")

    @functools.partial(
        pl.kernel,
        mesh=mesh,
        out_type=jax.ShapeDtypeStruct((B, DIM), jnp.float32),
        compiler_params=pltpu.CompilerParams(use_tc_tiling_on_sc=False),
        scratch_types=[
            pltpu.VMEM((CHUNK // SUB, SUB), jnp.int32),
            pltpu.VMEM((CHUNK, DIM), jnp.float32),
            pltpu.SemaphoreType.DMA,
        ],
    )
    def _k(table_hbm, idx_hbm, out_hbm, idx_v, rows_v, sem):
        wid = lax.axis_index("s") * NC + lax.axis_index("c")
        idx_row_base = wid * (per_w // SUB)
        out_base = wid * per_w

        def body(g, carry):
            pltpu.sync_copy(
                idx_hbm.at[pl.ds(idx_row_base + g * (CHUNK // SUB), CHUNK // SUB)],
                idx_v,
            )
            copies = [
                pltpu.async_copy(
                    table_hbm.at[idx_v.at[j]],
                    rows_v.at[pl.ds(j * SUB, SUB)],
                    sem,
                )
                for j in range(CHUNK // SUB)
            ]
            for c in copies:
                c.wait()
            pltpu.sync_copy(rows_v, out_hbm.at[pl.ds(out_base + g * CHUNK, CHUNK)])
            return carry

        lax.fori_loop(0, n_chunks, body, 0)

    return _k(table, idx2d)


def kernel(inputs, embeddings):
    shape = inputs.shape
    flat = jnp.reshape(inputs, (-1,)).astype(jnp.int32)
    B = flat.shape[0]
    per_w = B // NW
    n_chunks = per_w // CHUNK
    idx2d = jnp.reshape(flat, (B // SUB, SUB))
    out = _emb_lookup(embeddings, idx2d, B=B, per_w=per_w, n_chunks=n_chunks)
    return jnp.reshape(out, shape + (DIM,))
